# Initial kernel scaffold; baseline (speedup 1.0000x reference)
import jax
import jax.numpy as jnp
from jax import lax
from jax.experimental import pallas as pl
from jax.experimental.pallas import tpu as pltpu
from jax.experimental.pallas import tpu_sc as plsc

VOCAB = 1000000
SEQ = 8192
NSPIKE = 2730
NVREG = 171
SLICE = 62528
HALF = 31264
LAST_SIZE = 30816
LAST_OFF = 969184
GAIN_UP = 1.8
GAIN_DOWN = 0.6
KEY_TMASK = 1048575
INT_MAX = 2147483647


def _body(tok_hbm, out_hbm, toks, counts, spk, keys, cand_row, cand_all,
          cand_sh):
    c = lax.axis_index("c")
    s = lax.axis_index("s")
    w = 2 * s + c
    lo = s * SLICE
    iota = lax.broadcasted_iota(jnp.int32, (16,), 0)

    pltpu.sync_copy(tok_hbm, toks)

    zeros16 = jnp.zeros((16,), jnp.float32)

    def zero_body(i, carry):
        counts[pl.ds(i * 16, 16)] = zeros16
        return carry

    lax.fori_loop(0, SLICE // 16, zero_body, 0)

    ones16 = jnp.ones((16,), jnp.float32)
    lane_masks = [iota == l for l in range(16)]

    def cnt_body(j, carry):
        pos = jnp.minimum(48 * j + 3 * iota + 2, SEQ - 1)
        t = plsc.load_gather(toks, [pos])
        spk[pl.ds(j * 16, 16)] = t
        valid = (16 * j + iota) < NSPIKE
        rel = t - lo
        inr = valid & (rel >= 0) & (rel < SLICE)
        idx = jnp.where(inr, rel, 0)
        for lm in lane_masks:
            plsc.addupdate_scatter(counts, [idx], ones16, mask=inr & lm)
        return carry

    lax.fori_loop(0, NVREG, cnt_body, 0)

    def key_body(j, carry):
        t = spk[pl.ds(j * 16, 16)]
        valid = (16 * j + iota) < NSPIKE
        rel = t - lo
        inr = valid & (rel >= 0) & (rel < SLICE)
        idx = jnp.where(inr, rel, 0)
        cnt = plsc.load_gather(counts, [idx]).astype(jnp.int32)
        key = (jnp.minimum(cnt, 2047) << 20) | (KEY_TMASK - t)
        keys[pl.ds(j * 16, 16)] = jnp.where(inr, key, -1)
        return carry

    lax.fori_loop(0, NVREG, key_body, 0)

    neg16 = jnp.full((16,), -1, jnp.int32)
    winners = neg16
    prev = jnp.int32(INT_MAX)
    for r in range(5):
        def max_body(j, m, prev=prev):
            k = keys[pl.ds(j * 16, 16)]
            return jnp.maximum(m, jnp.where(k < prev, k, -1))

        best = jnp.max(lax.fori_loop(0, NVREG, max_body, neg16))
        winners = jnp.where(iota == r, best, winners)
        prev = best

    cand_row[...] = winners
    pltpu.sync_copy(cand_row, cand_sh.at[s])
    plsc.subcore_barrier()
    pltpu.sync_copy(cand_sh, cand_all)

    gwin = neg16
    gprev = jnp.int32(INT_MAX)
    for r in range(5):
        m = neg16
        for row in range(16):
            k = cand_all[row]
            m = jnp.maximum(m, jnp.where(k < gprev, k, -1))
        gbest = jnp.max(m)
        gwin = jnp.where(iota == r, gbest, gwin)
        gprev = gbest

    base_local = c * HALF

    def tf_body(i, carry):
        off = base_local + i * 16
        v = counts[pl.ds(off, 16)]
        counts[pl.ds(off, 16)] = jnp.where(
            v > 0.0, jnp.float32(GAIN_DOWN), jnp.float32(1.0))
        return carry

    lax.fori_loop(0, HALF // 16, tf_body, 0)

    glob_off = w * HALF
    n = jnp.where(w == 31, LAST_SIZE, HALF)
    gvalid = gwin > 0
    tw = KEY_TMASK - (gwin & KEY_TMASK)
    in_half = gvalid & (tw >= glob_off) & (tw < glob_off + n)
    widx = jnp.where(in_half, tw - lo, 0)
    plsc.store_scatter(counts, [widx],
                       jnp.full((16,), GAIN_UP, jnp.float32), mask=in_half)

    @pl.when(w < 31)
    def _dma_full():
        pltpu.sync_copy(counts.at[pl.ds(base_local, HALF)],
                        out_hbm.at[pl.ds(glob_off, HALF)])

    @pl.when(w == 31)
    def _dma_last():
        pltpu.sync_copy(counts.at[pl.ds(HALF, LAST_SIZE)],
                        out_hbm.at[pl.ds(LAST_OFF, LAST_SIZE)])


_spiking_attention_sc = pl.kernel(
    _body,
    out_type=jax.ShapeDtypeStruct((VOCAB,), jnp.float32),
    mesh=plsc.VectorSubcoreMesh(core_axis_name="c", subcore_axis_name="s"),
    scratch_types=[
        pltpu.VMEM((SEQ,), jnp.int32),
        pltpu.VMEM((SLICE,), jnp.float32),
        pltpu.VMEM((NVREG * 16,), jnp.int32),
        pltpu.VMEM((NVREG * 16,), jnp.int32),
        pltpu.VMEM((16,), jnp.int32),
        pltpu.VMEM((16, 16), jnp.int32),
        pltpu.VMEM_SHARED((16, 16), jnp.int32),
    ],
    name="spiking_attention_sc",
)


def kernel(token_ids, channel_weights):
    del channel_weights
    return _spiking_attention_sc(token_ids)


# speedup vs baseline: 47.3297x; 47.3297x over previous
import jax
import jax.numpy as jnp
from jax import lax
from jax.experimental import pallas as pl
from jax.experimental.pallas import tpu as pltpu
from jax.experimental.pallas import tpu_sc as plsc

VOCAB = 1000000
SEQ = 8192
NSPIKE = 2730
NVREG = 171
SLICE = 62528
HALF = 31264
LAST_SIZE = 30816
LAST_OFF = 969184
GAIN_UP = 1.8
GAIN_DOWN = 0.6
KEY_TMASK = 1048575
INT_MAX = 2147483647


def _body(tok_hbm, out_hbm, toks, counts, spk, keys, cand_row, cand_all,
          cand_sh, bar):
    c = lax.axis_index("c")
    s = lax.axis_index("s")
    w = 2 * s + c
    lo = s * SLICE
    iota = lax.broadcasted_iota(jnp.int32, (16,), 0)

    @pl.when(s == 0)
    def _init_bar():
        bar[0] = 0
        bar[1] = 0

    def sw_barrier(slot):
        plsc.fetch_and_add(bar.at[slot], 1, subcore_id=0)

        def cond(v):
            return v < 16

        def poll(v):
            return plsc.fetch_and_add(bar.at[slot], 0, subcore_id=0)

        lax.while_loop(cond, poll, jnp.int32(0))

    pltpu.sync_copy(tok_hbm, toks)

    zeros16 = jnp.zeros((16,), jnp.float32)

    def zero_body(i, carry):
        counts[pl.ds(i * 16, 16)] = zeros16
        return carry

    lax.fori_loop(0, SLICE // 16, zero_body, 0)

    ones16 = jnp.ones((16,), jnp.float32)
    lane_masks = [iota == l for l in range(16)]

    def cnt_body(j, carry):
        pos = jnp.minimum(48 * j + 3 * iota + 2, SEQ - 1)
        t = plsc.load_gather(toks, [pos])
        spk[pl.ds(j * 16, 16)] = t
        valid = (16 * j + iota) < NSPIKE
        rel = t - lo
        inr = valid & (rel >= 0) & (rel < SLICE)
        idx = jnp.where(inr, rel, 0)
        for lm in lane_masks:
            plsc.addupdate_scatter(counts, [idx], ones16, mask=inr & lm)
        return carry

    lax.fori_loop(0, NVREG, cnt_body, 0)

    def key_body(j, carry):
        t = spk[pl.ds(j * 16, 16)]
        valid = (16 * j + iota) < NSPIKE
        rel = t - lo
        inr = valid & (rel >= 0) & (rel < SLICE)
        idx = jnp.where(inr, rel, 0)
        cnt = plsc.load_gather(counts, [idx]).astype(jnp.int32)
        key = (jnp.minimum(cnt, 2047) << 20) | (KEY_TMASK - t)
        keys[pl.ds(j * 16, 16)] = jnp.where(inr, key, -1)
        return carry

    lax.fori_loop(0, NVREG, key_body, 0)

    neg16 = jnp.full((16,), -1, jnp.int32)
    winners = neg16
    prev = jnp.int32(INT_MAX)
    for r in range(5):
        def max_body(j, m, prev=prev):
            k = keys[pl.ds(j * 16, 16)]
            return jnp.maximum(m, jnp.where(k < prev, k, -1))

        best = jnp.max(lax.fori_loop(0, NVREG, max_body, neg16))
        winners = jnp.where(iota == r, best, winners)
        prev = best

    cand_row[...] = winners
    pltpu.sync_copy(cand_row, cand_sh.at[pl.ds(s * 16, 16)])
    sw_barrier(0)

    base_local = c * HALF

    def tf_body(i, carry):
        off = base_local + i * 16
        v = counts[pl.ds(off, 16)]
        counts[pl.ds(off, 16)] = jnp.where(
            v > 0.0, jnp.float32(GAIN_DOWN), jnp.float32(1.0))
        return carry

    lax.fori_loop(0, HALF // 16, tf_body, 0)

    sw_barrier(1)

    pltpu.sync_copy(cand_sh, cand_all)

    gwin = neg16
    gprev = jnp.int32(INT_MAX)
    for r in range(5):
        m = neg16
        for row in range(16):
            k = cand_all[pl.ds(row * 16, 16)]
            m = jnp.maximum(m, jnp.where(k < gprev, k, -1))
        gbest = jnp.max(m)
        gwin = jnp.where(iota == r, gbest, gwin)
        gprev = gbest

    glob_off = w * HALF
    n = jnp.where(w == 31, LAST_SIZE, HALF)
    gvalid = gwin > 0
    tw = KEY_TMASK - (gwin & KEY_TMASK)
    in_half = gvalid & (tw >= glob_off) & (tw < glob_off + n)
    widx = jnp.where(in_half, tw - lo, 0)
    plsc.store_scatter(counts, [widx],
                       jnp.full((16,), GAIN_UP, jnp.float32), mask=in_half)

    @pl.when(w < 31)
    def _dma_full():
        pltpu.sync_copy(counts.at[pl.ds(base_local, HALF)],
                        out_hbm.at[pl.ds(glob_off, HALF)])

    @pl.when(w == 31)
    def _dma_last():
        pltpu.sync_copy(counts.at[pl.ds(HALF, LAST_SIZE)],
                        out_hbm.at[pl.ds(LAST_OFF, LAST_SIZE)])


_spiking_attention_sc = pl.kernel(
    _body,
    out_type=jax.ShapeDtypeStruct((VOCAB,), jnp.float32),
    mesh=plsc.VectorSubcoreMesh(core_axis_name="c", subcore_axis_name="s"),
    scratch_types=[
        pltpu.VMEM((SEQ,), jnp.int32),
        pltpu.VMEM((SLICE,), jnp.float32),
        pltpu.VMEM((NVREG * 16,), jnp.int32),
        pltpu.VMEM((NVREG * 16,), jnp.int32),
        pltpu.VMEM((16,), jnp.int32),
        pltpu.VMEM((256,), jnp.int32),
        pltpu.VMEM_SHARED((256,), jnp.int32),
        pltpu.SMEM((2,), jnp.int32),
    ],
    compiler_params=pltpu.CompilerParams(needs_layout_passes=False),
    name="spiking_attention_sc",
)


def kernel(token_ids, channel_weights):
    del channel_weights
    return _spiking_attention_sc(token_ids)


# speedup vs baseline: 84.0950x; 1.7768x over previous
import jax
import jax.numpy as jnp
from jax import lax
from jax.experimental import pallas as pl
from jax.experimental.pallas import tpu as pltpu
from jax.experimental.pallas import tpu_sc as plsc

VOCAB = 1000000
SEQ = 8192
NSPIKE = 2730
NVREG = 171
SLICE = 62528
HALF = 31264
LAST_SIZE = 30816
LAST_OFF = 969184
GAIN_UP = 1.8
GAIN_DOWN = 0.6
KEY_TMASK = 1048575
INT_MAX = 2147483647
FILL_UNROLL = 4
TOP_UNROLL = 3


def _body(tok_hbm, out_hbm, toks, cells, spk, keys, cand_row, cand_all,
          cand_sh, bar):
    c = lax.axis_index("c")
    s = lax.axis_index("s")
    w = 2 * s + c
    lo = s * SLICE
    iota = lax.broadcasted_iota(jnp.int32, (16,), 0)

    @pl.when(s == 0)
    def _init_bar():
        bar[0] = 0
        bar[1] = 0

    def sw_barrier(slot):
        plsc.fetch_and_add(bar.at[slot], 1, subcore_id=0)

        def cond(v):
            return v < 16

        def poll(v):
            return plsc.fetch_and_add(bar.at[slot], 0, subcore_id=0)

        lax.while_loop(cond, poll, jnp.int32(0))

    pltpu.sync_copy(tok_hbm, toks)

    ones16f = jnp.ones((16,), jnp.float32)

    def fill_body(i, carry):
        for u in range(FILL_UNROLL):
            cells[pl.ds((i * FILL_UNROLL + u) * 16, 16)] = ones16f
        return carry

    lax.fori_loop(0, SLICE // 16 // FILL_UNROLL, fill_body, 0)

    def cnt_body(j, carry):
        pos = jnp.minimum(48 * j + 3 * iota + 2, SEQ - 1)
        t = plsc.load_gather(toks, [pos])
        spk[pl.ds(j * 16, 16)] = t
        valid = (16 * j + iota) < NSPIKE
        rel = t - lo
        inr = valid & (rel >= 0) & (rel < SLICE)
        idx = jnp.where(inr, rel, 0)
        plsc.addupdate_scatter(cells, [idx], ones16f, mask=inr)
        return carry

    lax.fori_loop(0, NVREG, cnt_body, 0)

    def key_body(j, carry):
        t = spk[pl.ds(j * 16, 16)]
        valid = (16 * j + iota) < NSPIKE
        rel = t - lo
        inr = valid & (rel >= 0) & (rel < SLICE)
        idx = jnp.where(inr, rel, 0)
        cnt = plsc.load_gather(cells, [idx]).astype(jnp.int32) - 1
        key = (jnp.minimum(cnt, 2047) << 20) | (KEY_TMASK - t)
        keys[pl.ds(j * 16, 16)] = jnp.where(inr, key, -1)
        return carry

    lax.fori_loop(0, NVREG, key_body, 0)

    neg16 = jnp.full((16,), -1, jnp.int32)
    winners = neg16
    prev = jnp.int32(INT_MAX)
    for r in range(5):
        def max_body(j, m, prev=prev):
            for u in range(TOP_UNROLL):
                k = keys[pl.ds((j * TOP_UNROLL + u) * 16, 16)]
                m = jnp.maximum(m, jnp.where(k < prev, k, -1))
            return m

        best = jnp.max(lax.fori_loop(0, NVREG // TOP_UNROLL, max_body, neg16))
        winners = jnp.where(iota == r, best, winners)
        prev = best

    cand_row[...] = winners
    pltpu.sync_copy(cand_row, cand_sh.at[pl.ds(s * 16, 16)])
    sw_barrier(0)

    point6 = jnp.full((16,), GAIN_DOWN, jnp.float32)

    def fix_body(j, carry):
        t = spk[pl.ds(j * 16, 16)]
        valid = (16 * j + iota) < NSPIKE
        rel = t - lo
        inr = valid & (rel >= 0) & (rel < SLICE)
        idx = jnp.where(inr, rel, 0)
        plsc.store_scatter(cells, [idx], point6, mask=inr)
        return carry

    lax.fori_loop(0, NVREG, fix_body, 0)

    sw_barrier(1)

    pltpu.sync_copy(cand_sh, cand_all)

    gwin = neg16
    gprev = jnp.int32(INT_MAX)
    for r in range(5):
        m = neg16
        for row in range(16):
            k = cand_all[pl.ds(row * 16, 16)]
            m = jnp.maximum(m, jnp.where(k < gprev, k, -1))
        gbest = jnp.max(m)
        gwin = jnp.where(iota == r, gbest, gwin)
        gprev = gbest

    base_local = c * HALF
    glob_off = w * HALF
    n = jnp.where(w == 31, LAST_SIZE, HALF)
    gvalid = gwin > 0
    tw = KEY_TMASK - (gwin & KEY_TMASK)
    in_half = gvalid & (tw >= glob_off) & (tw < glob_off + n)
    widx = jnp.where(in_half, tw - lo, 0)
    plsc.store_scatter(cells, [widx],
                       jnp.full((16,), GAIN_UP, jnp.float32), mask=in_half)

    @pl.when(w < 31)
    def _dma_full():
        pltpu.sync_copy(cells.at[pl.ds(base_local, HALF)],
                        out_hbm.at[pl.ds(glob_off, HALF)])

    @pl.when(w == 31)
    def _dma_last():
        pltpu.sync_copy(cells.at[pl.ds(HALF, LAST_SIZE)],
                        out_hbm.at[pl.ds(LAST_OFF, LAST_SIZE)])


_spiking_attention_sc = pl.kernel(
    _body,
    out_type=jax.ShapeDtypeStruct((VOCAB,), jnp.float32),
    mesh=plsc.VectorSubcoreMesh(core_axis_name="c", subcore_axis_name="s"),
    scratch_types=[
        pltpu.VMEM((SEQ,), jnp.int32),
        pltpu.VMEM((SLICE,), jnp.float32),
        pltpu.VMEM((NVREG * 16,), jnp.int32),
        pltpu.VMEM((NVREG * 16,), jnp.int32),
        pltpu.VMEM((16,), jnp.int32),
        pltpu.VMEM((256,), jnp.int32),
        pltpu.VMEM_SHARED((256,), jnp.int32),
        pltpu.SMEM((2,), jnp.int32),
    ],
    compiler_params=pltpu.CompilerParams(needs_layout_passes=False),
    name="spiking_attention_sc",
)


def kernel(token_ids, channel_weights):
    del channel_weights
    return _spiking_attention_sc(token_ids)


# speedup vs baseline: 84.9802x; 1.0105x over previous
import jax
import jax.numpy as jnp
from jax import lax
from jax.experimental import pallas as pl
from jax.experimental.pallas import tpu as pltpu
from jax.experimental.pallas import tpu_sc as plsc

VOCAB = 1000000
SEQ = 8192
NSPIKE = 2730
NVREG = 171
SLICE = 62528
HALF = 31264
LAST_SIZE = 30816
LAST_OFF = 969184
GAIN_UP = 1.8
GAIN_DOWN = 0.6
KEY_TMASK = 1048575
INT_MAX = 2147483647
FILL_UNROLL = 4
FILL_ITERS = 489
SLICE_PAD = 62592
TOP_UNROLL = 3


def _body(tok_hbm, out_hbm, toks, cells, spk, keys, cand_row, cand_all,
          vals18, cand_sh, bar):
    c = lax.axis_index("c")
    s = lax.axis_index("s")
    w = 2 * s + c
    lo = s * SLICE
    iota = lax.broadcasted_iota(jnp.int32, (16,), 0)

    @pl.when(s == 0)
    def _init_bar():
        bar[0] = 0
        bar[1] = 0

    def sw_barrier(slot):
        plsc.fetch_and_add(bar.at[slot], 1, subcore_id=0)

        def cond(v):
            return v < 16

        def poll(v):
            return plsc.fetch_and_add(bar.at[slot], 0, subcore_id=0)

        lax.while_loop(cond, poll, jnp.int32(0))

    pltpu.sync_copy(tok_hbm, toks)

    ones16f = jnp.ones((16,), jnp.float32)
    base_local = c * HALF

    def fill_body(i, carry):
        for u in range(FILL_UNROLL):
            cells[pl.ds(base_local + (i * FILL_UNROLL + u) * 16, 16)] = ones16f
        return carry

    lax.fori_loop(0, FILL_ITERS, fill_body, 0)

    def stage_body(j, carry):
        pos = jnp.minimum(48 * j + 3 * iota + 2, SEQ - 1)
        t = plsc.load_gather(toks, [pos])
        spk[pl.ds(j * 16, 16)] = t
        valid = (16 * j + iota) < NSPIKE
        rel = t - lo
        inr = valid & (rel >= 0) & (rel < SLICE)
        idx = jnp.where(inr, rel, 0)
        plsc.store_scatter(cells, [idx], ones16f, mask=inr)
        return carry

    lax.fori_loop(0, NVREG, stage_body, 0)

    def cnt_body(j, carry):
        t = spk[pl.ds(j * 16, 16)]
        valid = (16 * j + iota) < NSPIKE
        rel = t - lo
        inr = valid & (rel >= 0) & (rel < SLICE)
        idx = jnp.where(inr, rel, 0)
        plsc.addupdate_scatter(cells, [idx], ones16f, mask=inr)
        return carry

    lax.fori_loop(0, NVREG, cnt_body, 0)

    point6 = jnp.full((16,), GAIN_DOWN, jnp.float32)

    def key_body(j, carry):
        t = spk[pl.ds(j * 16, 16)]
        valid = (16 * j + iota) < NSPIKE
        rel = t - lo
        inr = valid & (rel >= 0) & (rel < SLICE)
        idx = jnp.where(inr, rel, 0)
        cnt = plsc.load_gather(cells, [idx]).astype(jnp.int32) - 1
        key = (jnp.minimum(cnt, 2047) << 20) | (KEY_TMASK - t)
        keys[pl.ds(j * 16, 16)] = jnp.where(inr, key, -1)
        plsc.store_scatter(cells, [idx], point6, mask=inr)
        return carry

    lax.fori_loop(0, NVREG, key_body, 0)

    neg16 = jnp.full((16,), -1, jnp.int32)
    winners = neg16
    prev = jnp.int32(INT_MAX)
    for r in range(5):
        def max_body(j, m, prev=prev):
            for u in range(TOP_UNROLL):
                k = keys[pl.ds((j * TOP_UNROLL + u) * 16, 16)]
                m = jnp.maximum(m, jnp.where(k < prev, k, -1))
            return m

        best = jnp.max(lax.fori_loop(0, NVREG // TOP_UNROLL, max_body, neg16))
        winners = jnp.where(iota == r, best, winners)
        prev = best

    cand_row[...] = winners
    pltpu.sync_copy(cand_row, cand_sh.at[pl.ds(s * 16, 16)])
    sw_barrier(0)

    glob_off = w * HALF

    @pl.when(w < 31)
    def _dma_full():
        pltpu.sync_copy(cells.at[pl.ds(base_local, HALF)],
                        out_hbm.at[pl.ds(glob_off, HALF)])

    @pl.when(w == 31)
    def _dma_last():
        pltpu.sync_copy(cells.at[pl.ds(HALF, LAST_SIZE)],
                        out_hbm.at[pl.ds(LAST_OFF, LAST_SIZE)])

    sw_barrier(1)

    pltpu.sync_copy(cand_sh, cand_all)

    gwin = neg16
    gprev = jnp.int32(INT_MAX)
    for r in range(5):
        m = neg16
        for row in range(16):
            k = cand_all[pl.ds(row * 16, 16)]
            m = jnp.maximum(m, jnp.where(k < gprev, k, -1))
        gbest = jnp.max(m)
        gwin = jnp.where(iota == r, gbest, gwin)
        gprev = gbest

    n = jnp.where(w == 31, LAST_SIZE, HALF)
    gvalid = gwin > 0
    tw = KEY_TMASK - (gwin & KEY_TMASK)
    in_half = gvalid & (tw >= glob_off) & (tw < glob_off + n)

    @pl.when(jnp.any(in_half))
    def _patch():
        fb = jnp.max(jnp.where(in_half, tw, -1))
        idx = jnp.where(in_half, tw, fb)
        vals18[...] = jnp.full((16,), GAIN_UP, jnp.float32)
        pltpu.sync_copy(vals18, out_hbm.at[idx])


_spiking_attention_sc = pl.kernel(
    _body,
    out_type=jax.ShapeDtypeStruct((VOCAB,), jnp.float32),
    mesh=plsc.VectorSubcoreMesh(core_axis_name="c", subcore_axis_name="s"),
    scratch_types=[
        pltpu.VMEM((SEQ,), jnp.int32),
        pltpu.VMEM((SLICE_PAD,), jnp.float32),
        pltpu.VMEM((NVREG * 16,), jnp.int32),
        pltpu.VMEM((NVREG * 16,), jnp.int32),
        pltpu.VMEM((16,), jnp.int32),
        pltpu.VMEM((256,), jnp.int32),
        pltpu.VMEM((16,), jnp.float32),
        pltpu.VMEM_SHARED((256,), jnp.int32),
        pltpu.SMEM((2,), jnp.int32),
    ],
    compiler_params=pltpu.CompilerParams(needs_layout_passes=False),
    name="spiking_attention_sc",
)


def kernel(token_ids, channel_weights):
    del channel_weights
    return _spiking_attention_sc(token_ids)


# speedup vs baseline: 86.6798x; 1.0200x over previous
import jax
import jax.numpy as jnp
from jax import lax
from jax.experimental import pallas as pl
from jax.experimental.pallas import tpu as pltpu
from jax.experimental.pallas import tpu_sc as plsc

VOCAB = 1000000
SEQ = 8192
NSPIKE = 2730
NVREG = 171
SLICE = 62528
HALF = 31264
LAST_SIZE = 30816
LAST_OFF = 969184
GAIN_UP = 1.8
GAIN_DOWN = 0.6
KEY_TMASK = 1048575
INT_MAX = 2147483647
FILL_UNROLL = 8
FILL_ITERS = 245
SLICE_PAD = 62656
TOP_UNROLL = 3


def _body(tok_hbm, out_hbm, toks, cells, spk, keys, cand_row, cand_all,
          vals18, cand_sh, bar):
    c = lax.axis_index("c")
    s = lax.axis_index("s")
    w = 2 * s + c
    lo = s * SLICE
    iota = lax.broadcasted_iota(jnp.int32, (16,), 0)

    @pl.when(s == 0)
    def _init_bar():
        bar[0] = 0
        bar[1] = 0

    def sw_barrier(slot):
        plsc.fetch_and_add(bar.at[slot], 1, subcore_id=0)

        def cond(v):
            return v < 16

        def poll(v):
            return plsc.fetch_and_add(bar.at[slot], 0, subcore_id=0)

        lax.while_loop(cond, poll, jnp.int32(0))

    pltpu.sync_copy(tok_hbm, toks)

    ones16f = jnp.ones((16,), jnp.float32)
    base_local = c * HALF

    def fill_body(i, carry):
        for u in range(FILL_UNROLL):
            cells[pl.ds(base_local + (i * FILL_UNROLL + u) * 16, 16)] = ones16f
        return carry

    lax.fori_loop(0, FILL_ITERS, fill_body, 0)

    def stage_body(j, carry):
        pos = jnp.minimum(48 * j + 3 * iota + 2, SEQ - 1)
        t = plsc.load_gather(toks, [pos])
        spk[pl.ds(j * 16, 16)] = t
        valid = (16 * j + iota) < NSPIKE
        rel = t - lo
        inr = valid & (rel >= 0) & (rel < SLICE)
        idx = jnp.where(inr, rel, 0)
        plsc.store_scatter(cells, [idx], ones16f, mask=inr)
        return carry

    lax.fori_loop(0, NVREG, stage_body, 0)

    def cnt_body(j, carry):
        t = spk[pl.ds(j * 16, 16)]
        valid = (16 * j + iota) < NSPIKE
        rel = t - lo
        inr = valid & (rel >= 0) & (rel < SLICE)
        idx = jnp.where(inr, rel, 0)
        plsc.addupdate_scatter(cells, [idx], ones16f, mask=inr)
        return carry

    lax.fori_loop(0, NVREG, cnt_body, 0)

    point6 = jnp.full((16,), GAIN_DOWN, jnp.float32)

    def key_body(j, carry):
        t = spk[pl.ds(j * 16, 16)]
        valid = (16 * j + iota) < NSPIKE
        rel = t - lo
        inr = valid & (rel >= 0) & (rel < SLICE)
        idx = jnp.where(inr, rel, 0)
        cnt = plsc.load_gather(cells, [idx]).astype(jnp.int32) - 1
        key = (jnp.minimum(cnt, 2047) << 20) | (KEY_TMASK - t)
        keys[pl.ds(j * 16, 16)] = jnp.where(inr, key, -1)
        plsc.store_scatter(cells, [idx], point6, mask=inr)
        return carry

    lax.fori_loop(0, NVREG, key_body, 0)

    neg16 = jnp.full((16,), -1, jnp.int32)
    winners = neg16
    prev = jnp.int32(INT_MAX)
    for r in range(5):
        def max_body(j, m, prev=prev):
            for u in range(TOP_UNROLL):
                k = keys[pl.ds((j * TOP_UNROLL + u) * 16, 16)]
                m = jnp.maximum(m, jnp.where(k < prev, k, -1))
            return m

        best = jnp.max(lax.fori_loop(0, NVREG // TOP_UNROLL, max_body, neg16))
        winners = jnp.where(iota == r, best, winners)
        prev = best

    cand_row[...] = winners
    pltpu.sync_copy(cand_row, cand_sh.at[pl.ds(s * 16, 16)])
    sw_barrier(0)

    glob_off = w * HALF

    @pl.when(w < 31)
    def _dma_full():
        pltpu.sync_copy(cells.at[pl.ds(base_local, HALF)],
                        out_hbm.at[pl.ds(glob_off, HALF)])

    @pl.when(w == 31)
    def _dma_last():
        pltpu.sync_copy(cells.at[pl.ds(HALF, LAST_SIZE)],
                        out_hbm.at[pl.ds(LAST_OFF, LAST_SIZE)])

    sw_barrier(1)

    pltpu.sync_copy(cand_sh, cand_all)

    gwin = neg16
    gprev = jnp.int32(INT_MAX)
    for r in range(5):
        def merge_body(row, m, gprev=gprev):
            k = cand_all[pl.ds(row * 16, 16)]
            return jnp.maximum(m, jnp.where(k < gprev, k, -1))

        gbest = jnp.max(lax.fori_loop(0, 16, merge_body, neg16))
        gwin = jnp.where(iota == r, gbest, gwin)
        gprev = gbest

    n = jnp.where(w == 31, LAST_SIZE, HALF)
    gvalid = gwin > 0
    tw = KEY_TMASK - (gwin & KEY_TMASK)
    in_half = gvalid & (tw >= glob_off) & (tw < glob_off + n)

    @pl.when(jnp.any(in_half))
    def _patch():
        fb = jnp.max(jnp.where(in_half, tw, -1))
        idx = jnp.where(in_half, tw, fb)
        vals18[...] = jnp.full((16,), GAIN_UP, jnp.float32)
        pltpu.sync_copy(vals18, out_hbm.at[idx])


_spiking_attention_sc = pl.kernel(
    _body,
    out_type=jax.ShapeDtypeStruct((VOCAB,), jnp.float32),
    mesh=plsc.VectorSubcoreMesh(core_axis_name="c", subcore_axis_name="s"),
    scratch_types=[
        pltpu.VMEM((SEQ,), jnp.int32),
        pltpu.VMEM((SLICE_PAD,), jnp.float32),
        pltpu.VMEM((NVREG * 16,), jnp.int32),
        pltpu.VMEM((NVREG * 16,), jnp.int32),
        pltpu.VMEM((16,), jnp.int32),
        pltpu.VMEM((256,), jnp.int32),
        pltpu.VMEM((16,), jnp.float32),
        pltpu.VMEM_SHARED((256,), jnp.int32),
        pltpu.SMEM((2,), jnp.int32),
    ],
    compiler_params=pltpu.CompilerParams(needs_layout_passes=False),
    name="spiking_attention_sc",
)


def kernel(token_ids, channel_weights):
    del channel_weights
    return _spiking_attention_sc(token_ids)


# speedup vs baseline: 87.2428x; 1.0065x over previous
import jax
import jax.numpy as jnp
from jax import lax
from jax.experimental import pallas as pl
from jax.experimental.pallas import tpu as pltpu
from jax.experimental.pallas import tpu_sc as plsc

VOCAB = 1000000
SEQ = 8192
NSPIKE = 2730
NVREG = 171
NVREG_PAD = 172
SLICE = 62528
HALF = 31264
LAST_SIZE = 30816
LAST_OFF = 969184
GAIN_UP = 1.8
GAIN_DOWN = 0.6
KEY_TMASK = 1048575
INT_MAX = 2147483647
FILL_UNROLL = 8
FILL_ITERS = 245
SLICE_PAD = 62656
TOP_UNROLL = 4
PASS_UNROLL = 2


def _body(tok_hbm, out_hbm, toks, cells, spk, keys, cand_row, cand_all,
          vals18, cand_sh, bar):
    c = lax.axis_index("c")
    s = lax.axis_index("s")
    w = 2 * s + c
    lo = s * SLICE
    iota = lax.broadcasted_iota(jnp.int32, (16,), 0)

    @pl.when(s == 0)
    def _init_bar():
        bar[0] = 0
        bar[1] = 0

    def sw_barrier(slot):
        plsc.fetch_and_add(bar.at[slot], 1, subcore_id=0)

        def cond(v):
            return v < 16

        def poll(v):
            return plsc.fetch_and_add(bar.at[slot], 0, subcore_id=0)

        lax.while_loop(cond, poll, jnp.int32(0))

    pltpu.sync_copy(tok_hbm, toks)

    ones16f = jnp.ones((16,), jnp.float32)
    base_local = c * HALF

    def fill_body(i, carry):
        for u in range(FILL_UNROLL):
            cells[pl.ds(base_local + (i * FILL_UNROLL + u) * 16, 16)] = ones16f
        return carry

    lax.fori_loop(0, FILL_ITERS, fill_body, 0)

    def stage_body(i, carry):
        for u in range(PASS_UNROLL):
            j = i * PASS_UNROLL + u
            pos = jnp.minimum(48 * j + 3 * iota + 2, SEQ - 1)
            t = plsc.load_gather(toks, [pos])
            spk[pl.ds(j * 16, 16)] = t
            valid = (16 * j + iota) < NSPIKE
            rel = t - lo
            inr = valid & (rel >= 0) & (rel < SLICE)
            idx = jnp.where(inr, rel, 0)
            plsc.store_scatter(cells, [idx], ones16f, mask=inr)
        return carry

    lax.fori_loop(0, NVREG_PAD // PASS_UNROLL, stage_body, 0)

    def cnt_body(i, carry):
        for u in range(PASS_UNROLL):
            j = i * PASS_UNROLL + u
            t = spk[pl.ds(j * 16, 16)]
            valid = (16 * j + iota) < NSPIKE
            rel = t - lo
            inr = valid & (rel >= 0) & (rel < SLICE)
            idx = jnp.where(inr, rel, 0)
            plsc.addupdate_scatter(cells, [idx], ones16f, mask=inr)
        return carry

    lax.fori_loop(0, NVREG_PAD // PASS_UNROLL, cnt_body, 0)

    point6 = jnp.full((16,), GAIN_DOWN, jnp.float32)
    neg16 = jnp.full((16,), -1, jnp.int32)

    def key_body(i, m0):
        for u in range(PASS_UNROLL):
            j = i * PASS_UNROLL + u
            t = spk[pl.ds(j * 16, 16)]
            valid = (16 * j + iota) < NSPIKE
            rel = t - lo
            inr = valid & (rel >= 0) & (rel < SLICE)
            idx = jnp.where(inr, rel, 0)
            cnt = plsc.load_gather(cells, [idx]).astype(jnp.int32) - 1
            key = jnp.where(inr, (jnp.minimum(cnt, 2047) << 20)
                            | (KEY_TMASK - t), -1)
            keys[pl.ds(j * 16, 16)] = key
            plsc.store_scatter(cells, [idx], point6, mask=inr)
            m0 = jnp.maximum(m0, key)
        return m0

    m0 = lax.fori_loop(0, NVREG_PAD // PASS_UNROLL, key_body, neg16)

    best = jnp.max(m0)
    winners = jnp.where(iota == 0, best, neg16)
    prev = best
    for r in range(1, 5):
        def max_body(j, m, prev=prev):
            for u in range(TOP_UNROLL):
                k = keys[pl.ds((j * TOP_UNROLL + u) * 16, 16)]
                m = jnp.maximum(m, jnp.where(k < prev, k, -1))
            return m

        best = jnp.max(
            lax.fori_loop(0, NVREG_PAD // TOP_UNROLL, max_body, neg16))
        winners = jnp.where(iota == r, best, winners)
        prev = best

    cand_row[...] = winners
    pltpu.sync_copy(cand_row, cand_sh.at[pl.ds(s * 16, 16)])
    sw_barrier(0)

    glob_off = w * HALF

    @pl.when(w < 31)
    def _dma_full():
        pltpu.sync_copy(cells.at[pl.ds(base_local, HALF)],
                        out_hbm.at[pl.ds(glob_off, HALF)])

    @pl.when(w == 31)
    def _dma_last():
        pltpu.sync_copy(cells.at[pl.ds(HALF, LAST_SIZE)],
                        out_hbm.at[pl.ds(LAST_OFF, LAST_SIZE)])

    sw_barrier(1)

    pltpu.sync_copy(cand_sh, cand_all)

    gwin = neg16
    gprev = jnp.int32(INT_MAX)
    for r in range(5):
        def merge_body(row, m, gprev=gprev):
            k = cand_all[pl.ds(row * 16, 16)]
            return jnp.maximum(m, jnp.where(k < gprev, k, -1))

        gbest = jnp.max(lax.fori_loop(0, 16, merge_body, neg16))
        gwin = jnp.where(iota == r, gbest, gwin)
        gprev = gbest

    n = jnp.where(w == 31, LAST_SIZE, HALF)
    gvalid = gwin > 0
    tw = KEY_TMASK - (gwin & KEY_TMASK)
    in_half = gvalid & (tw >= glob_off) & (tw < glob_off + n)

    @pl.when(jnp.any(in_half))
    def _patch():
        fb = jnp.max(jnp.where(in_half, tw, -1))
        idx = jnp.where(in_half, tw, fb)
        vals18[...] = jnp.full((16,), GAIN_UP, jnp.float32)
        pltpu.sync_copy(vals18, out_hbm.at[idx])


_spiking_attention_sc = pl.kernel(
    _body,
    out_type=jax.ShapeDtypeStruct((VOCAB,), jnp.float32),
    mesh=plsc.VectorSubcoreMesh(core_axis_name="c", subcore_axis_name="s"),
    scratch_types=[
        pltpu.VMEM((SEQ,), jnp.int32),
        pltpu.VMEM((SLICE_PAD,), jnp.float32),
        pltpu.VMEM((NVREG_PAD * 16,), jnp.int32),
        pltpu.VMEM((NVREG_PAD * 16,), jnp.int32),
        pltpu.VMEM((16,), jnp.int32),
        pltpu.VMEM((256,), jnp.int32),
        pltpu.VMEM((16,), jnp.float32),
        pltpu.VMEM_SHARED((256,), jnp.int32),
        pltpu.SMEM((2,), jnp.int32),
    ],
    compiler_params=pltpu.CompilerParams(needs_layout_passes=False),
    name="spiking_attention_sc",
)


def kernel(token_ids, channel_weights):
    del channel_weights
    return _spiking_attention_sc(token_ids)


# speedup vs baseline: 87.6189x; 1.0043x over previous
import jax
import jax.numpy as jnp
from jax import lax
from jax.experimental import pallas as pl
from jax.experimental.pallas import tpu as pltpu
from jax.experimental.pallas import tpu_sc as plsc

VOCAB = 1000000
SEQ = 8192
NSPIKE = 2730
NVREG = 171
NVREG_PAD = 172
SLICE = 62528
HALF = 31264
LAST_SIZE = 30816
LAST_OFF = 969184
GAIN_UP = 1.8
GAIN_DOWN = 0.6
KEY_TMASK = 1048575
INT_MAX = 2147483647
FILL_UNROLL = 8
FILL_ITERS = 245
SLICE_PAD = 62656
TOP_UNROLL = 4
PASS_UNROLL = 2


def _body(tok_hbm, out_hbm, toks, cells, spk, keys, cand_row, cand_all,
          vals18, cand_sh, bar):
    c = lax.axis_index("c")
    s = lax.axis_index("s")
    w = 2 * s + c
    lo = s * SLICE
    iota = lax.broadcasted_iota(jnp.int32, (16,), 0)

    @pl.when(s == 0)
    def _init_bar():
        bar[0] = 0
        bar[1] = 0

    def sw_barrier(slot):
        plsc.fetch_and_add(bar.at[slot], 1, subcore_id=0)

        def cond(v):
            return v < 16

        def poll(v):
            return plsc.fetch_and_add(bar.at[slot], 0, subcore_id=0)

        lax.while_loop(cond, poll, jnp.int32(0))

    pltpu.sync_copy(tok_hbm, toks)

    ones16f = jnp.ones((16,), jnp.float32)
    base_local = c * HALF

    @plsc.parallel_loop(0, FILL_ITERS * FILL_UNROLL, unroll=FILL_UNROLL)
    def fill_body(i):
        cells[pl.ds(base_local + i * 16, 16)] = ones16f

    def stage_body(i, carry):
        for u in range(PASS_UNROLL):
            j = i * PASS_UNROLL + u
            pos = jnp.minimum(48 * j + 3 * iota + 2, SEQ - 1)
            t = plsc.load_gather(toks, [pos])
            spk[pl.ds(j * 16, 16)] = t
            valid = (16 * j + iota) < NSPIKE
            rel = t - lo
            inr = valid & (rel >= 0) & (rel < SLICE)
            idx = jnp.where(inr, rel, 0)
            plsc.store_scatter(cells, [idx], ones16f, mask=inr)
        return carry

    lax.fori_loop(0, NVREG_PAD // PASS_UNROLL, stage_body, 0)

    def cnt_body(i, carry):
        for u in range(PASS_UNROLL):
            j = i * PASS_UNROLL + u
            t = spk[pl.ds(j * 16, 16)]
            valid = (16 * j + iota) < NSPIKE
            rel = t - lo
            inr = valid & (rel >= 0) & (rel < SLICE)
            idx = jnp.where(inr, rel, 0)
            plsc.addupdate_scatter(cells, [idx], ones16f, mask=inr)
        return carry

    lax.fori_loop(0, NVREG_PAD // PASS_UNROLL, cnt_body, 0)

    point6 = jnp.full((16,), GAIN_DOWN, jnp.float32)
    neg16 = jnp.full((16,), -1, jnp.int32)

    def key_body(i, m0):
        for u in range(PASS_UNROLL):
            j = i * PASS_UNROLL + u
            t = spk[pl.ds(j * 16, 16)]
            valid = (16 * j + iota) < NSPIKE
            rel = t - lo
            inr = valid & (rel >= 0) & (rel < SLICE)
            idx = jnp.where(inr, rel, 0)
            cnt = plsc.load_gather(cells, [idx]).astype(jnp.int32) - 1
            key = jnp.where(inr, (jnp.minimum(cnt, 2047) << 20)
                            | (KEY_TMASK - t), -1)
            keys[pl.ds(j * 16, 16)] = key
            plsc.store_scatter(cells, [idx], point6, mask=inr)
            m0 = jnp.maximum(m0, key)
        return m0

    m0 = lax.fori_loop(0, NVREG_PAD // PASS_UNROLL, key_body, neg16)

    best = jnp.max(m0)
    winners = jnp.where(iota == 0, best, neg16)
    prev = best
    for r in range(1, 5):
        def max_body(j, m, prev=prev):
            k = keys[pl.ds(j * 16, 16)]
            return jnp.maximum(m, jnp.where(k < prev, k, -1))

        m = plsc.parallel_loop(0, NVREG_PAD, unroll=TOP_UNROLL,
                               carry=neg16)(max_body)
        best = jnp.max(m)
        winners = jnp.where(iota == r, best, winners)
        prev = best

    cand_row[...] = winners
    pltpu.sync_copy(cand_row, cand_sh.at[pl.ds(s * 16, 16)])
    sw_barrier(0)

    glob_off = w * HALF

    @pl.when(w < 31)
    def _dma_full():
        pltpu.sync_copy(cells.at[pl.ds(base_local, HALF)],
                        out_hbm.at[pl.ds(glob_off, HALF)])

    @pl.when(w == 31)
    def _dma_last():
        pltpu.sync_copy(cells.at[pl.ds(HALF, LAST_SIZE)],
                        out_hbm.at[pl.ds(LAST_OFF, LAST_SIZE)])

    sw_barrier(1)

    pltpu.sync_copy(cand_sh, cand_all)

    gwin = neg16
    gprev = jnp.int32(INT_MAX)
    for r in range(5):
        def merge_body(row, m, gprev=gprev):
            k = cand_all[pl.ds(row * 16, 16)]
            return jnp.maximum(m, jnp.where(k < gprev, k, -1))

        gbest = jnp.max(lax.fori_loop(0, 16, merge_body, neg16))
        gwin = jnp.where(iota == r, gbest, gwin)
        gprev = gbest

    n = jnp.where(w == 31, LAST_SIZE, HALF)
    gvalid = gwin > 0
    tw = KEY_TMASK - (gwin & KEY_TMASK)
    in_half = gvalid & (tw >= glob_off) & (tw < glob_off + n)

    @pl.when(jnp.any(in_half))
    def _patch():
        fb = jnp.max(jnp.where(in_half, tw, -1))
        idx = jnp.where(in_half, tw, fb)
        vals18[...] = jnp.full((16,), GAIN_UP, jnp.float32)
        pltpu.sync_copy(vals18, out_hbm.at[idx])


_spiking_attention_sc = pl.kernel(
    _body,
    out_type=jax.ShapeDtypeStruct((VOCAB,), jnp.float32),
    mesh=plsc.VectorSubcoreMesh(core_axis_name="c", subcore_axis_name="s"),
    scratch_types=[
        pltpu.VMEM((SEQ,), jnp.int32),
        pltpu.VMEM((SLICE_PAD,), jnp.float32),
        pltpu.VMEM((NVREG_PAD * 16,), jnp.int32),
        pltpu.VMEM((NVREG_PAD * 16,), jnp.int32),
        pltpu.VMEM((16,), jnp.int32),
        pltpu.VMEM((256,), jnp.int32),
        pltpu.VMEM((16,), jnp.float32),
        pltpu.VMEM_SHARED((256,), jnp.int32),
        pltpu.SMEM((2,), jnp.int32),
    ],
    compiler_params=pltpu.CompilerParams(needs_layout_passes=False),
    name="spiking_attention_sc",
)


def kernel(token_ids, channel_weights):
    del channel_weights
    return _spiking_attention_sc(token_ids)
